# Initial kernel scaffold; baseline (speedup 1.0000x reference)
import jax
import jax.numpy as jnp
from jax.experimental import pallas as pl


def kernel(x, table):
    raise NotImplementedError("write your pallas kernel here")


# speedup vs baseline: 1.5680x; 1.5680x over previous
import jax
import jax.numpy as jnp
from jax.experimental import pallas as pl

_ROWS = 256


def _bcast_body(tab_ref, out_ref):
    out_ref[0] = tab_ref[...]


def kernel(x, table):
    batch = x.shape[0]
    n_rows, embed = table.shape
    return pl.pallas_call(
        _bcast_body,
        grid=(n_rows // _ROWS, batch),
        in_specs=[pl.BlockSpec((_ROWS, embed), lambda r, b: (r, 0))],
        out_specs=pl.BlockSpec((1, _ROWS, embed), lambda r, b: (b, r, 0)),
        out_shape=jax.ShapeDtypeStruct((batch, n_rows, embed), table.dtype),
    )(table)
